# baseline (speedup 1.0000x reference)
import functools

import jax
import jax.numpy as jnp
from jax import lax
from jax.experimental import pallas as pl
from jax.experimental.pallas import tpu as pltpu
from jax.experimental.pallas import tpu_sc as plsc

N_CLASSES = 100000
N_EMBS = 90000
N_MISS = N_CLASSES - N_EMBS
TEXT_DIM = 768
EMB_DIM = 64
BATCH = 16384
CW = 128
HALF_E = N_EMBS // 2
HALF_M = N_MISS // 2
N_PACKED = N_CLASSES // 2

NC, NS, L = 2, 16, 16
NW = NC * NS
B_PER_W = BATCH // NW
CH = 64
NCH = B_PER_W // CH

PB = 3000
MB = 1000


def _project(e, w, b):
    y = lax.dot_general(
        e, w, (((1,), (1,)), ((), ())), preferred_element_type=jnp.float32
    ) + b
    mean = jnp.mean(y, axis=1, keepdims=True)
    d = y - mean
    var = jnp.sum(d * d, axis=1, keepdims=True) * (1.0 / (EMB_DIM - 1))
    return y * lax.rsqrt(var)


def _proj_body(elo_ref, ehi_ref, w_ref, b_ref, out_ref):
    w = w_ref[...]
    b = b_ref[...]
    z_lo = _project(elo_ref[...], w, b)
    z_hi = _project(ehi_ref[...], w, b)
    out_ref[...] = jnp.concatenate([z_lo, z_hi], axis=1)


def _miss_body(mlo_ref, mhi_ref, c_ref, out_ref):
    del c_ref
    out_ref[...] = jnp.concatenate([mlo_ref[...], mhi_ref[...]], axis=1)


def _sc_mesh():
    return plsc.VectorSubcoreMesh(
        core_axis_name="c", subcore_axis_name="s", num_cores=NC, num_subcores=NS
    )


@functools.partial(
    pl.kernel,
    mesh=_sc_mesh(),
    out_type=[
        jax.ShapeDtypeStruct((BATCH, CW), jnp.float32),
        jax.ShapeDtypeStruct((NW, NCH, CH), jnp.float32),
        jax.ShapeDtypeStruct((NW, NCH, CH), jnp.int32),
    ],
    scratch_types=[
        pltpu.VMEM((NCH, CH), jnp.int32),
        pltpu.VMEM((NCH, CH), jnp.int32),
        pltpu.VMEM((NCH, CH), jnp.int32),
        pltpu.VMEM((NCH, CH), jnp.float32),
        pltpu.VMEM((B_PER_W, CW), jnp.float32),
        pltpu.SemaphoreType.DMA,
        pltpu.SemaphoreType.DMA,
        pltpu.SemaphoreType.DMA,
    ],
)
def _sc_gather(x_hbm, indmap_hbm, embmap_hbm, scales_hbm, comb_hbm,
               rows_out, scl_out, code_out,
               xv, indv, emv, sclv, gv, sem_g, sem_r, sem_w):
    wid = lax.axis_index("s") * NC + lax.axis_index("c")
    base = wid * B_PER_W

    for j in range(NCH):
        pltpu.sync_copy(x_hbm.at[pl.ds(base + j * CH, CH)], xv.at[j])
    descs = []
    for j in range(NCH):
        descs.append(pltpu.async_copy(indmap_hbm.at[xv.at[j]], indv.at[j], sem_g))
        descs.append(pltpu.async_copy(scales_hbm.at[xv.at[j]], sclv.at[j], sem_g))
        descs.append(pltpu.async_copy(embmap_hbm.at[xv.at[j]], emv.at[j], sem_g))
    for d in descs:
        d.wait()

    for j in range(NCH):
        for i in range(CH // L):
            sl = pl.ds(i * L, L)
            ind = indv[j, sl]
            miss = ind < 0
            s = jnp.maximum(ind, 0)
            rp = s >= HALF_E
            pp = jnp.where(rp, s - HALF_E, s)
            em = jnp.minimum(jnp.maximum(emv[j, sl], 0), N_MISS - 1)
            rm = em >= HALF_M
            pm = jnp.where(rm, em - HALF_M, em) + HALF_E
            p = jnp.where(miss, pm, pp)
            right = jnp.where(miss, rm, rp)
            indv[j, sl] = jnp.minimum(p, N_PACKED - 1)
            emv[j, sl] = (jnp.where(right, 1, 0)
                          + jnp.where(miss, 2, 0))
    d_code = pltpu.async_copy(emv, code_out.at[wid], sem_w)
    d_scl = pltpu.async_copy(sclv, scl_out.at[wid], sem_w)

    gdescs = []
    for j in range(NCH):
        gdescs.append(
            pltpu.async_copy(comb_hbm.at[indv.at[j]], gv.at[pl.ds(j * CH, CH)],
                             sem_r)
        )
    for d in gdescs:
        d.wait()
    pltpu.sync_copy(gv, rows_out.at[pl.ds(base, B_PER_W)])
    d_code.wait()
    d_scl.wait()


_OUT_BLK = 2048


def _final_body(g_ref, scl_ref, code_ref, out_ref):
    g = g_ref[...]
    code = code_ref[...]
    right = lax.rem(code, 2) == 1
    half = jnp.where(right, g[:, EMB_DIM:], g[:, :EMB_DIM])
    out_ref[...] = jnp.where(code >= 2, half, scl_ref[...] * half)


def kernel(x, indmap, embmap, embs, W1, b1, scales, missing_w):
    scales1 = scales.reshape(N_CLASSES)
    b2 = b1.reshape(1, EMB_DIM)

    comb = pl.pallas_call(
        _proj_body,
        grid=(HALF_E // PB,),
        in_specs=[
            pl.BlockSpec((PB, TEXT_DIM), lambda i: (i, 0)),
            pl.BlockSpec((PB, TEXT_DIM), lambda i: (HALF_E // PB + i, 0)),
            pl.BlockSpec((EMB_DIM, TEXT_DIM), lambda i: (0, 0)),
            pl.BlockSpec((1, EMB_DIM), lambda i: (0, 0)),
        ],
        out_specs=pl.BlockSpec((PB, CW), lambda i: (i, 0)),
        out_shape=jax.ShapeDtypeStruct((N_PACKED, CW), jnp.float32),
    )(embs, embs, W1, b2)

    comb = pl.pallas_call(
        _miss_body,
        grid=(HALF_M // MB,),
        in_specs=[
            pl.BlockSpec((MB, EMB_DIM), lambda i: (i, 0)),
            pl.BlockSpec((MB, EMB_DIM), lambda i: (HALF_M // MB + i, 0)),
            pl.BlockSpec((8, CW), lambda i: (0, 0)),
        ],
        out_specs=pl.BlockSpec((MB, CW), lambda i: (HALF_E // MB + i, 0)),
        out_shape=jax.ShapeDtypeStruct((N_PACKED, CW), jnp.float32),
        input_output_aliases={2: 0},
    )(missing_w, missing_w, comb)

    rows, scl, code = _sc_gather(x, indmap, embmap, scales1, comb)
    scl2 = scl.reshape(BATCH, 1)
    code2 = code.reshape(BATCH, 1)

    out = pl.pallas_call(
        _final_body,
        grid=(BATCH // _OUT_BLK,),
        in_specs=[
            pl.BlockSpec((_OUT_BLK, CW), lambda i: (i, 0)),
            pl.BlockSpec((_OUT_BLK, 1), lambda i: (i, 0)),
            pl.BlockSpec((_OUT_BLK, 1), lambda i: (i, 0)),
        ],
        out_specs=pl.BlockSpec((_OUT_BLK, EMB_DIM), lambda i: (i, 0)),
        out_shape=jax.ShapeDtypeStruct((BATCH, EMB_DIM), jnp.float32),
    )(rows, scl2, code2)
    return out


# speedup vs baseline: 1.0169x; 1.0169x over previous
import functools

import jax
import jax.numpy as jnp
from jax import lax
from jax.experimental import pallas as pl
from jax.experimental.pallas import tpu as pltpu
from jax.experimental.pallas import tpu_sc as plsc

N_CLASSES = 100000
N_EMBS = 90000
N_MISS = N_CLASSES - N_EMBS
TEXT_DIM = 768
EMB_DIM = 64
BATCH = 16384
CW = 128
HALF_E = N_EMBS // 2
HALF_M = N_MISS // 2
N_PACKED = N_CLASSES // 2

NC, NS, L = 2, 16, 16
NW = NC * NS
B_PER_W = BATCH // NW
CH = 64
NCH = B_PER_W // CH

PB = 1800
MB = 1000


def _project(e, w, b):
    y = lax.dot_general(
        e, w, (((1,), (1,)), ((), ())), preferred_element_type=jnp.float32
    ) + b
    mean = jnp.mean(y, axis=1, keepdims=True)
    d = y - mean
    var = jnp.sum(d * d, axis=1, keepdims=True) * (1.0 / (EMB_DIM - 1))
    return y * lax.rsqrt(var)


def _proj_body(elo_ref, ehi_ref, w_ref, b_ref, out_ref):
    w = w_ref[...]
    b = b_ref[...]
    z_lo = _project(elo_ref[...], w, b)
    z_hi = _project(ehi_ref[...], w, b)
    out_ref[...] = jnp.concatenate([z_lo, z_hi], axis=1)


def _miss_body(mlo_ref, mhi_ref, c_ref, out_ref):
    del c_ref
    out_ref[...] = jnp.concatenate([mlo_ref[...], mhi_ref[...]], axis=1)


def _sc_mesh():
    return plsc.VectorSubcoreMesh(
        core_axis_name="c", subcore_axis_name="s", num_cores=NC, num_subcores=NS
    )


@functools.partial(
    pl.kernel,
    mesh=_sc_mesh(),
    out_type=[
        jax.ShapeDtypeStruct((BATCH, CW), jnp.float32),
        jax.ShapeDtypeStruct((NW, NCH, CH), jnp.float32),
        jax.ShapeDtypeStruct((NW, NCH, CH), jnp.int32),
    ],
    scratch_types=[
        pltpu.VMEM((NCH, CH), jnp.int32),
        pltpu.VMEM((NCH, CH), jnp.int32),
        pltpu.VMEM((NCH, CH), jnp.int32),
        pltpu.VMEM((NCH, CH), jnp.float32),
        pltpu.VMEM((B_PER_W, CW), jnp.float32),
        pltpu.SemaphoreType.DMA,
        pltpu.SemaphoreType.DMA,
        pltpu.SemaphoreType.DMA,
    ],
)
def _sc_gather(x_hbm, indmap_hbm, embmap_hbm, scales_hbm, comb_hbm,
               rows_out, scl_out, code_out,
               xv, indv, emv, sclv, gv, sem_g, sem_r, sem_w):
    wid = lax.axis_index("s") * NC + lax.axis_index("c")
    base = wid * B_PER_W

    for j in range(NCH):
        pltpu.sync_copy(x_hbm.at[pl.ds(base + j * CH, CH)], xv.at[j])
    descs = []
    for j in range(NCH):
        descs.append(pltpu.async_copy(indmap_hbm.at[xv.at[j]], indv.at[j], sem_g))
        descs.append(pltpu.async_copy(scales_hbm.at[xv.at[j]], sclv.at[j], sem_g))
        descs.append(pltpu.async_copy(embmap_hbm.at[xv.at[j]], emv.at[j], sem_g))
    for d in descs:
        d.wait()

    for j in range(NCH):
        for i in range(CH // L):
            sl = pl.ds(i * L, L)
            ind = indv[j, sl]
            miss = ind < 0
            s = jnp.maximum(ind, 0)
            rp = s >= HALF_E
            pp = jnp.where(rp, s - HALF_E, s)
            em = jnp.minimum(jnp.maximum(emv[j, sl], 0), N_MISS - 1)
            rm = em >= HALF_M
            pm = jnp.where(rm, em - HALF_M, em) + HALF_E
            p = jnp.where(miss, pm, pp)
            right = jnp.where(miss, rm, rp)
            indv[j, sl] = jnp.minimum(p, N_PACKED - 1)
            emv[j, sl] = (jnp.where(right, 1, 0)
                          + jnp.where(miss, 2, 0))
    d_code = pltpu.async_copy(emv, code_out.at[wid], sem_w)
    d_scl = pltpu.async_copy(sclv, scl_out.at[wid], sem_w)

    gdescs = []
    for j in range(NCH):
        gdescs.append(
            pltpu.async_copy(comb_hbm.at[indv.at[j]], gv.at[pl.ds(j * CH, CH)],
                             sem_r)
        )
    for d in gdescs:
        d.wait()
    pltpu.sync_copy(gv, rows_out.at[pl.ds(base, B_PER_W)])
    d_code.wait()
    d_scl.wait()


_OUT_BLK = 2048


def _final_body(g_ref, scl_ref, code_ref, out_ref):
    g = g_ref[...]
    code = code_ref[...]
    right = lax.rem(code, 2) == 1
    half = jnp.where(right, g[:, EMB_DIM:], g[:, :EMB_DIM])
    out_ref[...] = jnp.where(code >= 2, half, scl_ref[...] * half)


def kernel(x, indmap, embmap, embs, W1, b1, scales, missing_w):
    scales1 = scales.reshape(N_CLASSES)
    b2 = b1.reshape(1, EMB_DIM)

    comb = pl.pallas_call(
        _proj_body,
        grid=(HALF_E // PB,),
        in_specs=[
            pl.BlockSpec((PB, TEXT_DIM), lambda i: (i, 0)),
            pl.BlockSpec((PB, TEXT_DIM), lambda i: (HALF_E // PB + i, 0)),
            pl.BlockSpec((EMB_DIM, TEXT_DIM), lambda i: (0, 0)),
            pl.BlockSpec((1, EMB_DIM), lambda i: (0, 0)),
        ],
        out_specs=pl.BlockSpec((PB, CW), lambda i: (i, 0)),
        out_shape=jax.ShapeDtypeStruct((N_PACKED, CW), jnp.float32),
    )(embs, embs, W1, b2)

    comb = pl.pallas_call(
        _miss_body,
        grid=(HALF_M // MB,),
        in_specs=[
            pl.BlockSpec((MB, EMB_DIM), lambda i: (i, 0)),
            pl.BlockSpec((MB, EMB_DIM), lambda i: (HALF_M // MB + i, 0)),
            pl.BlockSpec((8, CW), lambda i: (0, 0)),
        ],
        out_specs=pl.BlockSpec((MB, CW), lambda i: (HALF_E // MB + i, 0)),
        out_shape=jax.ShapeDtypeStruct((N_PACKED, CW), jnp.float32),
        input_output_aliases={2: 0},
    )(missing_w, missing_w, comb)

    rows, scl, code = _sc_gather(x, indmap, embmap, scales1, comb)
    scl2 = scl.reshape(BATCH, 1)
    code2 = code.reshape(BATCH, 1)

    out = pl.pallas_call(
        _final_body,
        grid=(BATCH // _OUT_BLK,),
        in_specs=[
            pl.BlockSpec((_OUT_BLK, CW), lambda i: (i, 0)),
            pl.BlockSpec((_OUT_BLK, 1), lambda i: (i, 0)),
            pl.BlockSpec((_OUT_BLK, 1), lambda i: (i, 0)),
        ],
        out_specs=pl.BlockSpec((_OUT_BLK, EMB_DIM), lambda i: (i, 0)),
        out_shape=jax.ShapeDtypeStruct((BATCH, EMB_DIM), jnp.float32),
    )(rows, scl2, code2)
    return out
